# baseline (speedup 1.0000x reference)
import functools

import numpy as np

import jax
import jax.numpy as jnp
from jax import lax
from jax.experimental import pallas as pl
from jax.experimental.pallas import tpu as pltpu
from jax.experimental.pallas import tpu_sc as plsc

N = 10000
E = 320000
D_IN = 128
D_H = 64

NC = 2
NS = 16
KB = 128
RPW = 80
EPAD = NC * NS * RPW * KB
NPAD = 10240

_mesh = plsc.VectorSubcoreMesh(core_axis_name="c", subcore_axis_name="s")
_SC_PARAMS = pltpu.CompilerParams(use_tc_tiling_on_sc=False)


_DEG_PIPE = 8


@functools.partial(
    pl.kernel,
    mesh=_mesh,
    out_type=jax.ShapeDtypeStruct((2 * NPAD,), jnp.float32),
    scratch_types=[
        pltpu.VMEM((RPW, KB), jnp.int32),
        pltpu.VMEM((KB,), jnp.float32),
        pltpu.VMEM((640,), jnp.float32),
        pltpu.VMEM_SHARED((NPAD,), jnp.float32),
        pltpu.SemaphoreType.DMA,
        pltpu.SemaphoreType.DMA,
    ],
    compiler_params=_SC_PARAMS,
)
def _deg_kernel(dst_hbm, out_hbm, dst_v, ones_v, z_v, acc_sh, isem, ssem):
    c = lax.axis_index("c")
    s = lax.axis_index("s")
    for i in range(640 // 16):
        z_v[pl.ds(i * 16, 16)] = jnp.zeros((16,), jnp.float32)
    for i in range(KB // 16):
        ones_v[pl.ds(i * 16, 16)] = jnp.ones((16,), jnp.float32)

    row0 = (c * NS + s) * RPW
    pltpu.async_copy(dst_hbm.at[pl.ds(row0, RPW)], dst_v, isem)
    pltpu.sync_copy(z_v, acc_sh.at[pl.ds(s * 640, 640)])
    pltpu.make_async_copy(dst_hbm.at[pl.ds(row0, RPW)], dst_v, isem).wait()
    plsc.subcore_barrier()

    for j in range(_DEG_PIPE):
        pltpu.async_copy(ones_v, acc_sh.at[dst_v.at[j]], ssem, add=True)

    def body(j, carry):
        pltpu.make_async_copy(ones_v, acc_sh.at[dst_v.at[j]], ssem).wait()
        pltpu.async_copy(ones_v, acc_sh.at[dst_v.at[j + _DEG_PIPE]], ssem,
                         add=True)
        return carry

    lax.fori_loop(0, RPW - _DEG_PIPE, body, 0)

    def tail(j, carry):
        pltpu.make_async_copy(ones_v, acc_sh.at[dst_v.at[j]], ssem).wait()
        return carry

    lax.fori_loop(RPW - _DEG_PIPE, RPW, tail, 0)
    plsc.subcore_barrier()
    pltpu.sync_copy(acc_sh.at[pl.ds(s * 640, 640)],
                    out_hbm.at[pl.ds(c * NPAD + s * 640, 640)])


GRP = 4
NGRP = RPW // GRP


@functools.partial(
    pl.kernel,
    mesh=_mesh,
    out_type=jax.ShapeDtypeStruct((2 * NPAD, D_H), jnp.float32),
    scratch_types=[
        pltpu.VMEM((RPW, KB), jnp.int32),
        pltpu.VMEM((RPW, KB), jnp.int32),
        pltpu.VMEM((2 * GRP, KB, D_H), jnp.float32),
        pltpu.VMEM((40, D_H), jnp.float32),
        pltpu.VMEM_SHARED((NPAD, D_H), jnp.float32),
        pltpu.SemaphoreType.DMA,
        pltpu.SemaphoreType.DMA,
        pltpu.SemaphoreType.DMA,
        pltpu.SemaphoreType.DMA,
        pltpu.SemaphoreType.DMA,
    ],
    compiler_params=_SC_PARAMS,
)
def _agg_kernel(y_hbm, src_hbm, dst_hbm, out_hbm,
                src_v, dst_v, rows_v, zb_v, acc_sh, gsem_a, gsem_b, ssem,
                zsem, isem):
    c = lax.axis_index("c")
    s = lax.axis_index("s")
    for i in range(40):
        for jj in range(D_H // 16):
            zb_v[i, pl.ds(jj * 16, 16)] = jnp.zeros((16,), jnp.float32)

    row0 = (c * NS + s) * RPW
    pltpu.async_copy(src_hbm.at[pl.ds(row0, RPW)], src_v, isem)
    pltpu.async_copy(dst_hbm.at[pl.ds(row0, RPW)], dst_v, isem)

    def zdst(k):
        return acc_sh.at[pl.ds(s * 640 + k * 40, 40)]

    for k in range(16):
        if k >= 8:
            pltpu.make_async_copy(zb_v, zdst(k - 8), zsem).wait()
        pltpu.async_copy(zb_v, zdst(k), zsem)
    for k in range(8, 16):
        pltpu.make_async_copy(zb_v, zdst(k), zsem).wait()
    pltpu.make_async_copy(src_hbm.at[pl.ds(row0, RPW)], src_v, isem).wait()
    pltpu.make_async_copy(dst_hbm.at[pl.ds(row0, RPW)], dst_v, isem).wait()
    plsc.subcore_barrier()

    def gather_start(r, b, sem):
        pltpu.async_copy(y_hbm.at[src_v.at[r]], rows_v.at[b], sem)

    def gather_wait(r, b, sem):
        pltpu.make_async_copy(y_hbm.at[src_v.at[r]], rows_v.at[b], sem).wait()

    def scatter_group(r0, b0):
        for b in range(GRP):
            pltpu.async_copy(rows_v.at[b0 + b], acc_sh.at[dst_v.at[r0 + b]],
                             ssem, add=True)
        for b in range(GRP):
            pltpu.make_async_copy(rows_v.at[b0 + b],
                                  acc_sh.at[dst_v.at[r0 + b]], ssem).wait()

    for b in range(GRP):
        gather_start(b, b, gsem_a)

    def body(t, carry):
        r0 = t * 2 * GRP
        for b in range(GRP):
            gather_start(r0 + GRP + b, GRP + b, gsem_b)
        for b in range(GRP):
            gather_wait(r0 + b, b, gsem_a)
        scatter_group(r0, 0)

        @pl.when(t < NGRP // 2 - 1)
        def _():
            for b in range(GRP):
                gather_start(r0 + 2 * GRP + b, b, gsem_a)

        for b in range(GRP):
            gather_wait(r0 + GRP + b, GRP + b, gsem_b)
        scatter_group(r0 + GRP, GRP)
        return carry

    lax.fori_loop(0, NGRP // 2, body, 0)
    plsc.subcore_barrier()
    pltpu.sync_copy(acc_sh.at[pl.ds(s * 640, 640)],
                    out_hbm.at[pl.ds(c * NPAD + s * 640, 640)])


_P = lax.Precision.DEFAULT
_DN = (((1,), (0,)), ((), ()))


def _dinv_body(degp_ref, dinv_ref):
    deg = degp_ref[0:NPAD // KB, :] + degp_ref[NPAD // KB:2 * NPAD // KB, :] + 1.0
    dinv_ref[...] = 1.0 / jnp.sqrt(deg)


def _y1_body(xp_ref, w1_ref, dinvp_ref, y_ref):
    xw = lax.dot_general(xp_ref[...], w1_ref[...], _DN, precision=_P,
                         preferred_element_type=jnp.float32)
    y_ref[...] = xw * dinvp_ref[...]


def _tc_mid_body(accp_ref, yp_ref, dinvp_ref, b_ref, w_ref, out_ref):
    dinv = dinvp_ref[...]
    acc = accp_ref[0:N // 2, :] + accp_ref[NPAD // 2:(NPAD + N) // 2, :] + yp_ref[...]
    h = jnp.maximum(acc * dinv + b_ref[...], 0.0)
    hw = lax.dot_general(h, w_ref[...], _DN, precision=_P,
                         preferred_element_type=jnp.float32)
    out_ref[...] = hw * dinv


def _tc_final_body(accp_ref, yp_ref, dinvp_ref, b3_ref, wd1_ref, bd1_ref,
                   wd2_ref, bd2_ref, wa_ref, ba_ref, h_ref, rec_ref, an_ref):
    acc = accp_ref[0:N // 2, :] + accp_ref[NPAD // 2:(NPAD + N) // 2, :] + yp_ref[...]
    h = jnp.maximum(acc * dinvp_ref[...] + b3_ref[...], 0.0)
    h_ref[...] = h
    t = jnp.maximum(
        lax.dot_general(h, wd1_ref[...], _DN, precision=_P,
                        preferred_element_type=jnp.float32) + bd1_ref[...], 0.0)
    rec_ref[...] = lax.dot_general(t, wd2_ref[...], _DN, precision=_P,
                                   preferred_element_type=jnp.float32) + bd2_ref[...]
    an_ref[...] = lax.dot_general(h, wa_ref[...], _DN, precision=_P,
                                  preferred_element_type=jnp.float32) + ba_ref[...]


_PAD_SRC = np.zeros((EPAD - E,), np.int32)
_PAD_DST = np.full((EPAD - E,), N, np.int32)


def _blkdiag(w):
    z = jnp.zeros_like(w)
    return jnp.concatenate(
        [jnp.concatenate([w, z], axis=1), jnp.concatenate([z, w], axis=1)],
        axis=0)


def _pairb(b):
    return jnp.concatenate([b, b]).reshape(1, -1)


def kernel(x, edge_index, W1, b1, W2, b2, W3, b3, Wd1, bd1, Wd2, bd2, Wa, ba):
    ei = edge_index.astype(jnp.int32)
    src2 = jnp.concatenate([ei[0], _PAD_SRC]).reshape(EPAD // KB, KB)
    dst2 = jnp.concatenate([ei[1], _PAD_DST]).reshape(EPAD // KB, KB)

    degp = _deg_kernel(dst2).reshape(2 * NPAD // KB, KB)
    dinv_lin = pl.pallas_call(
        _dinv_body,
        out_shape=jax.ShapeDtypeStruct((NPAD // KB, KB), jnp.float32),
    )(degp)
    dinvp = jnp.broadcast_to(
        dinv_lin.reshape(NPAD)[:N].reshape(N // 2, 2, 1),
        (N // 2, 2, D_H)).reshape(N // 2, 2 * D_H)

    xp = x.reshape(N // 2, 2 * D_IN)
    y1p = pl.pallas_call(
        _y1_body,
        out_shape=jax.ShapeDtypeStruct((N // 2, 2 * D_H), jnp.float32),
    )(xp, _blkdiag(W1), dinvp)

    acc1 = _agg_kernel(y1p.reshape(N, D_H), src2, dst2).reshape(NPAD, 2 * D_H)
    y2p = pl.pallas_call(
        _tc_mid_body,
        out_shape=jax.ShapeDtypeStruct((N // 2, 2 * D_H), jnp.float32),
    )(acc1, y1p, dinvp, _pairb(b1), _blkdiag(W2))

    acc2 = _agg_kernel(y2p.reshape(N, D_H), src2, dst2).reshape(NPAD, 2 * D_H)
    y3p = pl.pallas_call(
        _tc_mid_body,
        out_shape=jax.ShapeDtypeStruct((N // 2, 2 * D_H), jnp.float32),
    )(acc2, y2p, dinvp, _pairb(b2), _blkdiag(W3))

    acc3 = _agg_kernel(y3p.reshape(N, D_H), src2, dst2).reshape(NPAD, 2 * D_H)
    hp, recp, anp = pl.pallas_call(
        _tc_final_body,
        out_shape=(jax.ShapeDtypeStruct((N // 2, 2 * D_H), jnp.float32),
                   jax.ShapeDtypeStruct((N // 2, 2 * D_IN), jnp.float32),
                   jax.ShapeDtypeStruct((N // 2, 2), jnp.float32)),
    )(acc3, y3p, dinvp, _pairb(b3), _blkdiag(Wd1), _pairb(bd1),
      _blkdiag(Wd2), _pairb(bd2), _blkdiag(Wa), _pairb(ba))
    return (hp.reshape(N, D_H), recp.reshape(N, D_IN), anp.reshape(N, 1))


# speedup vs baseline: 3.2360x; 3.2360x over previous
import functools

import numpy as np

import jax
import jax.numpy as jnp
from jax import lax
from jax.experimental import pallas as pl
from jax.experimental.pallas import tpu as pltpu
from jax.experimental.pallas import tpu_sc as plsc

N = 10000
E = 320000
D_IN = 128
D_H = 64

NC = 2
NS = 16
KB = 128
RPW = 80
EPAD = NC * NS * RPW * KB
NPAD = 10240

_mesh = plsc.VectorSubcoreMesh(core_axis_name="c", subcore_axis_name="s")
_SC_PARAMS = pltpu.CompilerParams(use_tc_tiling_on_sc=False)


_DEG_PIPE = 8


@functools.partial(
    pl.kernel,
    mesh=_mesh,
    out_type=jax.ShapeDtypeStruct((2 * NPAD,), jnp.float32),
    scratch_types=[
        pltpu.VMEM((RPW, KB), jnp.int32),
        pltpu.VMEM((KB,), jnp.float32),
        pltpu.VMEM((640,), jnp.float32),
        pltpu.VMEM_SHARED((NPAD,), jnp.float32),
        pltpu.SemaphoreType.DMA,
        pltpu.SemaphoreType.DMA,
    ],
    compiler_params=_SC_PARAMS,
)
def _deg_kernel(dst_hbm, out_hbm, dst_v, ones_v, z_v, acc_sh, isem, ssem):
    c = lax.axis_index("c")
    s = lax.axis_index("s")
    for i in range(640 // 16):
        z_v[pl.ds(i * 16, 16)] = jnp.zeros((16,), jnp.float32)
    for i in range(KB // 16):
        ones_v[pl.ds(i * 16, 16)] = jnp.ones((16,), jnp.float32)

    row0 = (c * NS + s) * RPW
    pltpu.async_copy(dst_hbm.at[pl.ds(row0, RPW)], dst_v, isem)
    pltpu.sync_copy(z_v, acc_sh.at[pl.ds(s * 640, 640)])
    pltpu.make_async_copy(dst_hbm.at[pl.ds(row0, RPW)], dst_v, isem).wait()
    plsc.subcore_barrier()

    for j in range(_DEG_PIPE):
        pltpu.async_copy(ones_v, acc_sh.at[dst_v.at[j]], ssem, add=True)

    def body(j, carry):
        pltpu.make_async_copy(ones_v, acc_sh.at[dst_v.at[j]], ssem).wait()
        pltpu.async_copy(ones_v, acc_sh.at[dst_v.at[j + _DEG_PIPE]], ssem,
                         add=True)
        return carry

    lax.fori_loop(0, RPW - _DEG_PIPE, body, 0)

    def tail(j, carry):
        pltpu.make_async_copy(ones_v, acc_sh.at[dst_v.at[j]], ssem).wait()
        return carry

    lax.fori_loop(RPW - _DEG_PIPE, RPW, tail, 0)
    plsc.subcore_barrier()
    pltpu.sync_copy(acc_sh.at[pl.ds(s * 640, 640)],
                    out_hbm.at[pl.ds(c * NPAD + s * 640, 640)])


GRP = 4
NGRP = RPW // GRP


@functools.partial(
    pl.kernel,
    mesh=_mesh,
    out_type=jax.ShapeDtypeStruct((2 * NPAD, D_H), jnp.float32),
    scratch_types=[
        pltpu.VMEM((RPW, KB), jnp.int32),
        pltpu.VMEM((RPW, KB), jnp.int32),
        pltpu.VMEM((2 * GRP, KB, D_H), jnp.float32),
        pltpu.VMEM((40, D_H), jnp.float32),
        pltpu.VMEM_SHARED((NPAD, D_H), jnp.float32),
        pltpu.SemaphoreType.DMA,
        pltpu.SemaphoreType.DMA,
        pltpu.SemaphoreType.DMA,
        pltpu.SemaphoreType.DMA,
        pltpu.SemaphoreType.DMA,
    ],
    compiler_params=_SC_PARAMS,
)
def _agg_kernel(y_hbm, src_hbm, dst_hbm, out_hbm,
                src_v, dst_v, rows_v, zb_v, acc_sh, gsem_a, gsem_b, ssem,
                zsem, isem):
    c = lax.axis_index("c")
    s = lax.axis_index("s")
    for i in range(40):
        for jj in range(D_H // 16):
            zb_v[i, pl.ds(jj * 16, 16)] = jnp.zeros((16,), jnp.float32)

    row0 = (c * NS + s) * RPW
    pltpu.async_copy(src_hbm.at[pl.ds(row0, RPW)], src_v, isem)
    pltpu.async_copy(dst_hbm.at[pl.ds(row0, RPW)], dst_v, isem)

    def zdst(k):
        return acc_sh.at[pl.ds(s * 640 + k * 40, 40)]

    for k in range(16):
        if k >= 8:
            pltpu.make_async_copy(zb_v, zdst(k - 8), zsem).wait()
        pltpu.async_copy(zb_v, zdst(k), zsem)
    for k in range(8, 16):
        pltpu.make_async_copy(zb_v, zdst(k), zsem).wait()
    pltpu.make_async_copy(src_hbm.at[pl.ds(row0, RPW)], src_v, isem).wait()
    pltpu.make_async_copy(dst_hbm.at[pl.ds(row0, RPW)], dst_v, isem).wait()
    plsc.subcore_barrier()

    def gather_start(r, b, sem):
        pltpu.async_copy(y_hbm.at[src_v.at[r]], rows_v.at[b], sem)

    def gather_wait(r, b, sem):
        pltpu.make_async_copy(y_hbm.at[src_v.at[r]], rows_v.at[b], sem).wait()

    def scatter_group(r0, b0):
        for b in range(GRP):
            pltpu.async_copy(rows_v.at[b0 + b], acc_sh.at[dst_v.at[r0 + b]],
                             ssem, add=True)
        for b in range(GRP):
            pltpu.make_async_copy(rows_v.at[b0 + b],
                                  acc_sh.at[dst_v.at[r0 + b]], ssem).wait()

    for b in range(GRP):
        gather_start(b, b, gsem_a)

    def body(t, carry):
        r0 = t * 2 * GRP
        for b in range(GRP):
            gather_start(r0 + GRP + b, GRP + b, gsem_b)
        for b in range(GRP):
            gather_wait(r0 + b, b, gsem_a)
        scatter_group(r0, 0)

        @pl.when(t < NGRP // 2 - 1)
        def _():
            for b in range(GRP):
                gather_start(r0 + 2 * GRP + b, b, gsem_a)

        for b in range(GRP):
            gather_wait(r0 + GRP + b, GRP + b, gsem_b)
        scatter_group(r0 + GRP, GRP)
        return carry

    lax.fori_loop(0, NGRP // 2, body, 0)
    plsc.subcore_barrier()
    pltpu.sync_copy(acc_sh.at[pl.ds(s * 640, 640)],
                    out_hbm.at[pl.ds(c * NPAD + s * 640, 640)])


_P = lax.Precision.DEFAULT
_DN = (((1,), (0,)), ((), ()))


def _dinv_body(degp_ref, dinv_ref):
    deg = degp_ref[0:NPAD // KB, :] + degp_ref[NPAD // KB:2 * NPAD // KB, :] + 1.0
    dinv_ref[...] = 1.0 / jnp.sqrt(deg)


def _y1_body(xp_ref, w1_ref, dinvp_ref, y_ref):
    xw = lax.dot_general(xp_ref[...], w1_ref[...], _DN, precision=_P,
                         preferred_element_type=jnp.float32)
    y_ref[...] = xw * dinvp_ref[...]


def _tc_mid_body(accp_ref, yp_ref, dinvp_ref, b_ref, w_ref, out_ref):
    dinv = dinvp_ref[...]
    acc = accp_ref[0:N // 2, :] + accp_ref[NPAD // 2:(NPAD + N) // 2, :] + yp_ref[...]
    h = jnp.maximum(acc * dinv + b_ref[...], 0.0)
    hw = lax.dot_general(h, w_ref[...], _DN, precision=_P,
                         preferred_element_type=jnp.float32)
    out_ref[...] = hw * dinv


def _tc_final_body(accp_ref, yp_ref, dinvp_ref, b3_ref, wd1_ref, bd1_ref,
                   wd2_ref, bd2_ref, wa_ref, ba_ref, h_ref, rec_ref, an_ref):
    acc = accp_ref[0:N // 2, :] + accp_ref[NPAD // 2:(NPAD + N) // 2, :] + yp_ref[...]
    h = jnp.maximum(acc * dinvp_ref[...] + b3_ref[...], 0.0)
    h_ref[...] = h
    t = jnp.maximum(
        lax.dot_general(h, wd1_ref[...], _DN, precision=_P,
                        preferred_element_type=jnp.float32) + bd1_ref[...], 0.0)
    rec_ref[...] = lax.dot_general(t, wd2_ref[...], _DN, precision=_P,
                                   preferred_element_type=jnp.float32) + bd2_ref[...]
    an_ref[...] = lax.dot_general(h, wa_ref[...], _DN, precision=_P,
                                  preferred_element_type=jnp.float32) + ba_ref[...]


_PAD_SRC = (np.arange(EPAD - E) % N).astype(np.int32)
_PAD_DST = (N + np.arange(EPAD - E) % (NPAD - N)).astype(np.int32)


def _blkdiag(w):
    z = jnp.zeros_like(w)
    return jnp.concatenate(
        [jnp.concatenate([w, z], axis=1), jnp.concatenate([z, w], axis=1)],
        axis=0)


def _pairb(b):
    return jnp.concatenate([b, b]).reshape(1, -1)


def kernel(x, edge_index, W1, b1, W2, b2, W3, b3, Wd1, bd1, Wd2, bd2, Wa, ba):
    ei = edge_index.astype(jnp.int32)
    src2 = jnp.concatenate([ei[0], _PAD_SRC]).reshape(EPAD // KB, KB)
    dst2 = jnp.concatenate([ei[1], _PAD_DST]).reshape(EPAD // KB, KB)

    degp = _deg_kernel(dst2).reshape(2 * NPAD // KB, KB)
    dinv_lin = pl.pallas_call(
        _dinv_body,
        out_shape=jax.ShapeDtypeStruct((NPAD // KB, KB), jnp.float32),
    )(degp)
    dinvp = jnp.broadcast_to(
        dinv_lin.reshape(NPAD)[:N].reshape(N // 2, 2, 1),
        (N // 2, 2, D_H)).reshape(N // 2, 2 * D_H)

    xp = x.reshape(N // 2, 2 * D_IN)
    y1p = pl.pallas_call(
        _y1_body,
        out_shape=jax.ShapeDtypeStruct((N // 2, 2 * D_H), jnp.float32),
    )(xp, _blkdiag(W1), dinvp)

    acc1 = _agg_kernel(y1p.reshape(N, D_H), src2, dst2).reshape(NPAD, 2 * D_H)
    y2p = pl.pallas_call(
        _tc_mid_body,
        out_shape=jax.ShapeDtypeStruct((N // 2, 2 * D_H), jnp.float32),
    )(acc1, y1p, dinvp, _pairb(b1), _blkdiag(W2))

    acc2 = _agg_kernel(y2p.reshape(N, D_H), src2, dst2).reshape(NPAD, 2 * D_H)
    y3p = pl.pallas_call(
        _tc_mid_body,
        out_shape=jax.ShapeDtypeStruct((N // 2, 2 * D_H), jnp.float32),
    )(acc2, y2p, dinvp, _pairb(b2), _blkdiag(W3))

    acc3 = _agg_kernel(y3p.reshape(N, D_H), src2, dst2).reshape(NPAD, 2 * D_H)
    hp, recp, anp = pl.pallas_call(
        _tc_final_body,
        out_shape=(jax.ShapeDtypeStruct((N // 2, 2 * D_H), jnp.float32),
                   jax.ShapeDtypeStruct((N // 2, 2 * D_IN), jnp.float32),
                   jax.ShapeDtypeStruct((N // 2, 2), jnp.float32)),
    )(acc3, y3p, dinvp, _pairb(b3), _blkdiag(Wd1), _pairb(bd1),
      _blkdiag(Wd2), _pairb(bd2), _blkdiag(Wa), _pairb(ba))
    return (hp.reshape(N, D_H), recp.reshape(N, D_IN), anp.reshape(N, 1))
